# baseline (speedup 1.0000x reference)
import functools

import jax
import jax.numpy as jnp
from jax.experimental import pallas as pl
from jax.experimental.pallas import tpu as pltpu

H = 6
B = 4
S = 64
N = B * S
PREC = jax.lax.Precision.DEFAULT


def _dot(a, b):
    return jax.lax.dot_general(
        a, b, (((1,), (0,)), ((), ())),
        preferred_element_type=jnp.float32, precision=PREC)


def _dot_t(a, b):
    return jax.lax.dot_general(
        a, b, (((1,), (1,)), ((), ())),
        preferred_element_type=jnp.float32, precision=PREC)


def _ln(x, s, b):
    m = jnp.mean(x, -1, keepdims=True)
    v = jnp.mean((x - m) ** 2, -1, keepdims=True)
    return (x - m) / jnp.sqrt(v + 1e-5) * s + b


def _attend(q, k, v, dh):
    scale = dh ** -0.5
    outs = []
    for h in range(H):
        qh = q[:, h * dh:(h + 1) * dh]
        kh = k[:, h * dh:(h + 1) * dh]
        vh = v[:, h * dh:(h + 1) * dh]
        s = _dot_t(qh, kh) * scale
        rows = []
        for b in range(B):
            sb = s[b * S:(b + 1) * S, b * S:(b + 1) * S]
            sb = sb - jnp.max(sb, -1, keepdims=True)
            e = jnp.exp(sb)
            a = e / jnp.sum(e, -1, keepdims=True)
            rows.append(_dot(a, vh[b * S:(b + 1) * S, :]))
        outs.append(jnp.concatenate(rows, 0))
    return jnp.concatenate(outs, -1)


def _layer_body(xq_ref, xkv_ref, ls1_ref, lb1_ref, wq_ref, wk_ref, wv_ref,
                wo_ref, ls2_ref, lb2_ref, w1_ref, b1_ref, w2_ref, b2_ref,
                o_ref, xn_ref, *, dh, ht, hid):
    t = pl.program_id(0)

    @pl.when(t == 0)
    def _():
        qn = _ln(xq_ref[:], ls1_ref[:], lb1_ref[:])
        kvn = _ln(xkv_ref[:], ls1_ref[:], lb1_ref[:])
        q = _dot(qn, wq_ref[:])
        k = _dot(kvn, wk_ref[:])
        v = _dot(kvn, wv_ref[:])
        o = _attend(q, k, v, dh)
        x = xq_ref[:] + _dot(o, wo_ref[:])
        xn_ref[:] = _ln(x, ls2_ref[:], lb2_ref[:])
        o_ref[:] = x + b2_ref[:]

    @pl.when(t > 0)
    def _():
        h = jnp.maximum(_dot(xn_ref[:], w1_ref[:]) + b1_ref[:], 0.0)
        w2 = w2_ref[:]
        if hid % ht:
            off = (t - 1) * ht
            col = off + jax.lax.broadcasted_iota(jnp.int32, h.shape, 1)
            h = jnp.where(col < hid, h, 0.0)
            row = off + jax.lax.broadcasted_iota(jnp.int32, w2.shape, 0)
            w2 = jnp.where(row < hid, w2, 0.0)
        o_ref[:] += _dot(h, w2)


def _layer_call(xq, xkv, p):
    d = xq.shape[-1]
    hid = p['W1'].shape[-1]
    ht = 512
    nh = pl.cdiv(hid, ht)

    def w1m(t):
        return (0, jnp.maximum(t - 1, 0))

    def b1m(t):
        return (0, jnp.maximum(t - 1, 0))

    def w2m(t):
        return (jnp.maximum(t - 1, 0), 0)

    cst = lambda t: (0, 0)
    full = pl.BlockSpec((N, d), cst)
    vec = pl.BlockSpec((1, d), cst)
    return pl.pallas_call(
        functools.partial(_layer_body, dh=d // H, ht=ht, hid=hid),
        grid=(1 + nh,),
        in_specs=[full, full, vec, vec,
                  pl.BlockSpec((d, d), cst), pl.BlockSpec((d, d), cst),
                  pl.BlockSpec((d, d), cst), pl.BlockSpec((d, d), cst),
                  vec, vec,
                  pl.BlockSpec((d, ht), w1m), pl.BlockSpec((1, ht), b1m),
                  pl.BlockSpec((ht, d), w2m), vec],
        out_specs=pl.BlockSpec((N, d), cst),
        out_shape=jax.ShapeDtypeStruct((N, d), jnp.float32),
        scratch_shapes=[pltpu.VMEM((N, d), jnp.float32)],
    )(xq, xkv, p['ln1_s'].reshape(1, d), p['ln1_b'].reshape(1, d),
      p['Wq'], p['Wk'], p['Wv'], p['Wo'],
      p['ln2_s'].reshape(1, d), p['ln2_b'].reshape(1, d),
      p['W1'], p['b1'].reshape(1, hid), p['W2'], p['b2'].reshape(1, d))


def _qkv_body(x_ref, ls_ref, lb_ref, wq_ref, wk_ref, wv_ref,
              q_ref, k_ref, v_ref, xn_ref):
    t = pl.program_id(0)

    @pl.when(t == 0)
    def _():
        xn_ref[:] = _ln(x_ref[:], ls_ref[:], lb_ref[:])

    xn = xn_ref[:]
    q_ref[:] = _dot(xn, wq_ref[:])
    k_ref[:] = _dot(xn, wk_ref[:])
    v_ref[:] = _dot(xn, wv_ref[:])


def _qkv_call(x, p, ct=512):
    d = x.shape[-1]
    nt = pl.cdiv(d, ct)
    wspec = pl.BlockSpec((d, ct), lambda t: (0, t))
    ospec = pl.BlockSpec((N, ct), lambda t: (0, t))
    full = pl.BlockSpec((N, d), lambda t: (0, 0))
    vec = pl.BlockSpec((1, d), lambda t: (0, 0))
    return pl.pallas_call(
        _qkv_body,
        grid=(nt,),
        in_specs=[full, vec, vec, wspec, wspec, wspec],
        out_specs=(ospec, ospec, ospec),
        out_shape=tuple(jax.ShapeDtypeStruct((N, d), jnp.float32)
                        for _ in range(3)),
        scratch_shapes=[pltpu.VMEM((N, d), jnp.float32)],
    )(x, p['ln1_s'].reshape(1, d), p['ln1_b'].reshape(1, d),
      p['Wq'], p['Wk'], p['Wv'])


def _attn_out_body(q_ref, k_ref, v_ref, wo_ref, res_ref, o_ref, *, dh):
    o = _attend(q_ref[:], k_ref[:], v_ref[:], dh)
    o_ref[:] = res_ref[:] + _dot(o, wo_ref[:])


def _attn_out_call(q, k, v, wo, res):
    d = q.shape[-1]
    return pl.pallas_call(
        functools.partial(_attn_out_body, dh=d // H),
        out_shape=jax.ShapeDtypeStruct((N, d), jnp.float32),
    )(q, k, v, wo, res)


def _ffn_body(x_ref, ls_ref, lb_ref, w1a_ref, w1b_ref, b1_ref, w2a_ref,
              w2b_ref, b2_ref, o_ref, xn_ref, *, ht, hid, nsub):
    t = pl.program_id(0)

    @pl.when(t == 0)
    def _():
        xn_ref[:] = _ln(x_ref[:], ls_ref[:], lb_ref[:])
        o_ref[:] = x_ref[:] + b2_ref[:]

    xn = xn_ref[:]
    acc = None
    for i, (w1_ref, w2_ref) in enumerate(((w1a_ref, w2a_ref),
                                          (w1b_ref, w2b_ref))):
        b1 = b1_ref[:, i * ht:(i + 1) * ht]
        h = jnp.maximum(_dot(xn, w1_ref[:]) + b1, 0.0)
        w2 = w2_ref[:]
        col = t * 2 * ht + i * ht + jax.lax.broadcasted_iota(
            jnp.int32, h.shape, 1)
        h = jnp.where(col < hid, h, 0.0)
        row = t * 2 * ht + i * ht + jax.lax.broadcasted_iota(
            jnp.int32, w2.shape, 0)
        w2 = jnp.where(row < hid, w2, 0.0)
        part = _dot(h, w2)
        acc = part if acc is None else acc + part
    o_ref[:] += acc


def _ffn_call(x, p, ht):
    d = x.shape[-1]
    hid = p['W1'].shape[-1]
    nsub = pl.cdiv(hid, ht)
    nt = pl.cdiv(nsub, 2)
    last = nsub - 1

    def wa(t):
        return (0, jnp.minimum(2 * t, last))

    def wb(t):
        return (0, jnp.minimum(2 * t + 1, last))

    def w2a(t):
        return (jnp.minimum(2 * t, last), 0)

    def w2b(t):
        return (jnp.minimum(2 * t + 1, last), 0)

    return pl.pallas_call(
        functools.partial(_ffn_body, ht=ht, hid=hid, nsub=nsub),
        grid=(nt,),
        in_specs=[
            pl.BlockSpec((N, d), lambda t: (0, 0)),
            pl.BlockSpec((1, d), lambda t: (0, 0)),
            pl.BlockSpec((1, d), lambda t: (0, 0)),
            pl.BlockSpec((d, ht), wa),
            pl.BlockSpec((d, ht), wb),
            pl.BlockSpec((1, 2 * ht), lambda t: (0, t)),
            pl.BlockSpec((ht, d), w2a),
            pl.BlockSpec((ht, d), w2b),
            pl.BlockSpec((1, d), lambda t: (0, 0)),
        ],
        out_specs=pl.BlockSpec((N, d), lambda t: (0, 0)),
        out_shape=jax.ShapeDtypeStruct((N, d), jnp.float32),
        scratch_shapes=[pltpu.VMEM((N, d), jnp.float32)],
    )(x, p['ln2_s'].reshape(1, d), p['ln2_b'].reshape(1, d),
      p['W1'], p['W1'], p['b1'].reshape(1, hid), p['W2'], p['W2'],
      p['b2'].reshape(1, d))


def _layer_big(xq, xkv, p):
    q, k, v = _qkv_call(xq, p)
    x = _attn_out_call(q, k, v, p['Wo'], xq)
    return _ffn_call(x, p, ht=768)


def _mm_body(*refs, relu, has_bias, has_res):
    x_ref, w_ref = refs[0], refs[1]
    rest = list(refs[2:-1])
    o_ref = refs[-1]
    acc = _dot(x_ref[:], w_ref[:])
    if has_bias:
        acc = acc + rest.pop(0)[:]
    if has_res:
        acc = acc + rest.pop(0)[:]
    if relu:
        acc = jnp.maximum(acc, 0.0)
    o_ref[:] = acc


def _mm_call(x, w, bias=None, res=None, relu=False, ct=None):
    d = x.shape[-1]
    nout = w.shape[-1]
    ct = ct or nout
    nt = pl.cdiv(nout, ct)
    args = [x, w]
    specs = [pl.BlockSpec((N, d), lambda t: (0, 0)),
             pl.BlockSpec((d, ct), lambda t: (0, t))]
    if bias is not None:
        args.append(bias.reshape(1, nout))
        specs.append(pl.BlockSpec((1, ct), lambda t: (0, t)))
    if res is not None:
        args.append(res)
        specs.append(pl.BlockSpec((N, ct), lambda t: (0, t)))
    return pl.pallas_call(
        functools.partial(_mm_body, relu=relu, has_bias=bias is not None,
                          has_res=res is not None),
        grid=(nt,),
        in_specs=specs,
        out_specs=pl.BlockSpec((N, ct), lambda t: (0, t)),
        out_shape=jax.ShapeDtypeStruct((N, nout), jnp.float32),
    )(*args)


def _layer(xq, xkv, p):
    if xq.shape[-1] <= 600:
        return _layer_call(xq, xq if xkv is None else xkv, p)
    return _layer_big(xq, xq if xkv is None else xkv, p)


def _encoder(x, kv, plist):
    for p in plist:
        x = _layer(x, kv, p)
    return x


def kernel(x_l, x_a, x_v, seq_lengths, params):
    P = params
    xl = x_l.reshape(N, -1)
    xa = x_a.reshape(N, -1)
    xv = x_v.reshape(N, -1)

    h_a_l = _encoder(xa, xl, P['a_l'])
    h_a_v = _encoder(xa, xv, P['a_v'])
    h_as = _encoder(jnp.concatenate([h_a_l, h_a_v], -1), None, P['a_mem'])
    h_v_l = _encoder(xv, xl, P['v_l'])
    h_v_a = _encoder(xv, xa, P['v_a'])
    h_vs = _encoder(jnp.concatenate([h_v_l, h_v_a], -1), None, P['v_mem'])

    x_l_ext = _mm_call(xl, P['W_lext'], bias=P['b_lext'])
    last_hs = jnp.concatenate([x_l_ext, h_as, h_vs], -1)
    z = _mm_call(last_hs, P['W_p1'], bias=P['b_p1'], relu=True, ct=512)
    z = _mm_call(z, P['W_p1'], bias=P['b_p1'], ct=512)
    proj = _encoder(z, None, P['last'])
    out = _mm_call(proj, P['W_out'], bias=P['b_out'])

    last_hs_proj = proj.reshape(B, S, -1).transpose(1, 0, 2)
    output = out.reshape(B, S, -1).transpose(1, 0, 2)
    return last_hs_proj, output


# speedup vs baseline: 1.0785x; 1.0785x over previous
import functools

import jax
import jax.numpy as jnp
from jax.experimental import pallas as pl
from jax.experimental.pallas import tpu as pltpu

H = 6
B = 4
S = 64
N = B * S
PREC = jax.lax.Precision.DEFAULT


def _dot(a, b):
    return jax.lax.dot_general(
        a, b, (((1,), (0,)), ((), ())),
        preferred_element_type=jnp.float32, precision=PREC)


def _dot_t(a, b):
    return jax.lax.dot_general(
        a, b, (((1,), (1,)), ((), ())),
        preferred_element_type=jnp.float32, precision=PREC)


def _ln(x, s, b):
    m = jnp.mean(x, -1, keepdims=True)
    v = jnp.mean((x - m) ** 2, -1, keepdims=True)
    return (x - m) / jnp.sqrt(v + 1e-5) * s + b


def _attend(q, k, v, dh):
    scale = dh ** -0.5
    outs = []
    for h in range(H):
        qh = q[:, h * dh:(h + 1) * dh]
        kh = k[:, h * dh:(h + 1) * dh]
        vh = v[:, h * dh:(h + 1) * dh]
        s = _dot_t(qh, kh) * scale
        rows = []
        for b in range(B):
            sb = s[b * S:(b + 1) * S, b * S:(b + 1) * S]
            sb = sb - jnp.max(sb, -1, keepdims=True)
            e = jnp.exp(sb)
            a = e / jnp.sum(e, -1, keepdims=True)
            rows.append(_dot(a, vh[b * S:(b + 1) * S, :]))
        outs.append(jnp.concatenate(rows, 0))
    return jnp.concatenate(outs, -1)


def _layer_body(xq_ref, xkv_ref, pv_ref, wq_ref, wk_ref, wv_ref,
                wo_ref, w1_ref, b1_ref, w2_ref, o_ref, *, dh):
    ls1 = pv_ref[0:1, :]
    lb1 = pv_ref[1:2, :]
    ls2 = pv_ref[2:3, :]
    lb2 = pv_ref[3:4, :]
    b2 = pv_ref[4:5, :]
    qn = _ln(xq_ref[:], ls1, lb1)
    kvn = _ln(xkv_ref[:], ls1, lb1)
    q = _dot(qn, wq_ref[:])
    k = _dot(kvn, wk_ref[:])
    v = _dot(kvn, wv_ref[:])
    o = _attend(q, k, v, dh)
    x = xq_ref[:] + _dot(o, wo_ref[:])
    xn = _ln(x, ls2, lb2)
    h = jnp.maximum(_dot(xn, w1_ref[:]) + b1_ref[:], 0.0)
    o_ref[:] = x + _dot(h, w2_ref[:]) + b2


def _layer_call(xq, xkv, p):
    d = xq.shape[-1]
    hid = p['W1'].shape[-1]
    pv = jnp.stack([p['ln1_s'], p['ln1_b'], p['ln2_s'], p['ln2_b'],
                    p['b2']], 0)
    return pl.pallas_call(
        functools.partial(_layer_body, dh=d // H),
        out_shape=jax.ShapeDtypeStruct((N, d), jnp.float32),
    )(xq, xkv, pv, p['Wq'], p['Wk'], p['Wv'], p['Wo'],
      p['W1'], p['b1'].reshape(1, hid), p['W2'])


def _qkv_body(x_ref, ls_ref, lb_ref, wq_ref, wk_ref, wv_ref,
              q_ref, k_ref, v_ref, xn_ref):
    t = pl.program_id(0)

    @pl.when(t == 0)
    def _():
        xn_ref[:] = _ln(x_ref[:], ls_ref[:], lb_ref[:])

    xn = xn_ref[:]
    q_ref[:] = _dot(xn, wq_ref[:])
    k_ref[:] = _dot(xn, wk_ref[:])
    v_ref[:] = _dot(xn, wv_ref[:])


def _qkv_call(x, p, ct=512):
    d = x.shape[-1]
    nt = pl.cdiv(d, ct)
    wspec = pl.BlockSpec((d, ct), lambda t: (0, t))
    ospec = pl.BlockSpec((N, ct), lambda t: (0, t))
    full = pl.BlockSpec((N, d), lambda t: (0, 0))
    vec = pl.BlockSpec((1, d), lambda t: (0, 0))
    return pl.pallas_call(
        _qkv_body,
        grid=(nt,),
        in_specs=[full, vec, vec, wspec, wspec, wspec],
        out_specs=(ospec, ospec, ospec),
        out_shape=tuple(jax.ShapeDtypeStruct((N, d), jnp.float32)
                        for _ in range(3)),
        scratch_shapes=[pltpu.VMEM((N, d), jnp.float32)],
    )(x, p['ln1_s'].reshape(1, d), p['ln1_b'].reshape(1, d),
      p['Wq'], p['Wk'], p['Wv'])


def _attn_out_body(q_ref, k_ref, v_ref, wo_ref, res_ref, o_ref, *, dh):
    o = _attend(q_ref[:], k_ref[:], v_ref[:], dh)
    o_ref[:] = res_ref[:] + _dot(o, wo_ref[:])


def _attn_out_call(q, k, v, wo, res):
    d = q.shape[-1]
    return pl.pallas_call(
        functools.partial(_attn_out_body, dh=d // H),
        out_shape=jax.ShapeDtypeStruct((N, d), jnp.float32),
    )(q, k, v, wo, res)


def _ffn_body(x_ref, ls_ref, lb_ref, w1a_ref, w1b_ref, b1_ref, w2a_ref,
              w2b_ref, b2_ref, o_ref, xn_ref, *, ht, hid, nsub):
    t = pl.program_id(0)

    @pl.when(t == 0)
    def _():
        xn_ref[:] = _ln(x_ref[:], ls_ref[:], lb_ref[:])
        o_ref[:] = x_ref[:] + b2_ref[:]

    xn = xn_ref[:]
    acc = None
    for i, (w1_ref, w2_ref) in enumerate(((w1a_ref, w2a_ref),
                                          (w1b_ref, w2b_ref))):
        b1 = b1_ref[:, i * ht:(i + 1) * ht]
        h = jnp.maximum(_dot(xn, w1_ref[:]) + b1, 0.0)
        w2 = w2_ref[:]
        col = t * 2 * ht + i * ht + jax.lax.broadcasted_iota(
            jnp.int32, h.shape, 1)
        h = jnp.where(col < hid, h, 0.0)
        row = t * 2 * ht + i * ht + jax.lax.broadcasted_iota(
            jnp.int32, w2.shape, 0)
        w2 = jnp.where(row < hid, w2, 0.0)
        part = _dot(h, w2)
        acc = part if acc is None else acc + part
    o_ref[:] += acc


def _ffn_call(x, p, ht):
    d = x.shape[-1]
    hid = p['W1'].shape[-1]
    nsub = pl.cdiv(hid, ht)
    nt = pl.cdiv(nsub, 2)
    last = nsub - 1

    def wa(t):
        return (0, jnp.minimum(2 * t, last))

    def wb(t):
        return (0, jnp.minimum(2 * t + 1, last))

    def w2a(t):
        return (jnp.minimum(2 * t, last), 0)

    def w2b(t):
        return (jnp.minimum(2 * t + 1, last), 0)

    return pl.pallas_call(
        functools.partial(_ffn_body, ht=ht, hid=hid, nsub=nsub),
        grid=(nt,),
        in_specs=[
            pl.BlockSpec((N, d), lambda t: (0, 0)),
            pl.BlockSpec((1, d), lambda t: (0, 0)),
            pl.BlockSpec((1, d), lambda t: (0, 0)),
            pl.BlockSpec((d, ht), wa),
            pl.BlockSpec((d, ht), wb),
            pl.BlockSpec((1, 2 * ht), lambda t: (0, t)),
            pl.BlockSpec((ht, d), w2a),
            pl.BlockSpec((ht, d), w2b),
            pl.BlockSpec((1, d), lambda t: (0, 0)),
        ],
        out_specs=pl.BlockSpec((N, d), lambda t: (0, 0)),
        out_shape=jax.ShapeDtypeStruct((N, d), jnp.float32),
        scratch_shapes=[pltpu.VMEM((N, d), jnp.float32)],
    )(x, p['ln2_s'].reshape(1, d), p['ln2_b'].reshape(1, d),
      p['W1'], p['W1'], p['b1'].reshape(1, hid), p['W2'], p['W2'],
      p['b2'].reshape(1, d))


def _layer_big(xq, xkv, p):
    q, k, v = _qkv_call(xq, p)
    x = _attn_out_call(q, k, v, p['Wo'], xq)
    return _ffn_call(x, p, ht=768)


def _mm_body(*refs, relu, has_bias, has_res):
    x_ref, w_ref = refs[0], refs[1]
    rest = list(refs[2:-1])
    o_ref = refs[-1]
    acc = _dot(x_ref[:], w_ref[:])
    if has_bias:
        acc = acc + rest.pop(0)[:]
    if has_res:
        acc = acc + rest.pop(0)[:]
    if relu:
        acc = jnp.maximum(acc, 0.0)
    o_ref[:] = acc


def _mm_call(x, w, bias=None, res=None, relu=False, ct=None):
    d = x.shape[-1]
    nout = w.shape[-1]
    ct = ct or nout
    nt = pl.cdiv(nout, ct)
    args = [x, w]
    specs = [pl.BlockSpec((N, d), lambda t: (0, 0)),
             pl.BlockSpec((d, ct), lambda t: (0, t))]
    if bias is not None:
        args.append(bias.reshape(1, nout))
        specs.append(pl.BlockSpec((1, ct), lambda t: (0, t)))
    if res is not None:
        args.append(res)
        specs.append(pl.BlockSpec((N, ct), lambda t: (0, t)))
    return pl.pallas_call(
        functools.partial(_mm_body, relu=relu, has_bias=bias is not None,
                          has_res=res is not None),
        grid=(nt,),
        in_specs=specs,
        out_specs=pl.BlockSpec((N, ct), lambda t: (0, t)),
        out_shape=jax.ShapeDtypeStruct((N, nout), jnp.float32),
    )(*args)


def _layer(xq, xkv, p):
    if xq.shape[-1] <= 600:
        return _layer_call(xq, xq if xkv is None else xkv, p)
    return _layer_big(xq, xq if xkv is None else xkv, p)


def _encoder(x, kv, plist):
    for p in plist:
        x = _layer(x, kv, p)
    return x


def kernel(x_l, x_a, x_v, seq_lengths, params):
    P = params
    xl = x_l.reshape(N, -1)
    xa = x_a.reshape(N, -1)
    xv = x_v.reshape(N, -1)

    h_a_l = _encoder(xa, xl, P['a_l'])
    h_a_v = _encoder(xa, xv, P['a_v'])
    h_as = _encoder(jnp.concatenate([h_a_l, h_a_v], -1), None, P['a_mem'])
    h_v_l = _encoder(xv, xl, P['v_l'])
    h_v_a = _encoder(xv, xa, P['v_a'])
    h_vs = _encoder(jnp.concatenate([h_v_l, h_v_a], -1), None, P['v_mem'])

    x_l_ext = _mm_call(xl, P['W_lext'], bias=P['b_lext'])
    last_hs = jnp.concatenate([x_l_ext, h_as, h_vs], -1)
    z = _mm_call(last_hs, P['W_p1'], bias=P['b_p1'], relu=True, ct=512)
    z = _mm_call(z, P['W_p1'], bias=P['b_p1'], ct=512)
    proj = _encoder(z, None, P['last'])
    out = _mm_call(proj, P['W_out'], bias=P['b_out'])

    last_hs_proj = proj.reshape(B, S, -1).transpose(1, 0, 2)
    output = out.reshape(B, S, -1).transpose(1, 0, 2)
    return last_hs_proj, output


# speedup vs baseline: 1.0850x; 1.0060x over previous
import functools

import jax
import jax.numpy as jnp
from jax.experimental import pallas as pl
from jax.experimental.pallas import tpu as pltpu

H = 6
B = 4
S = 64
N = B * S
PREC = jax.lax.Precision.DEFAULT


def _dot(a, b):
    return jax.lax.dot_general(
        a, b, (((1,), (0,)), ((), ())),
        preferred_element_type=jnp.float32, precision=PREC)


def _dot_t(a, b):
    return jax.lax.dot_general(
        a, b, (((1,), (1,)), ((), ())),
        preferred_element_type=jnp.float32, precision=PREC)


def _ln(x, s, b):
    m = jnp.mean(x, -1, keepdims=True)
    v = jnp.mean((x - m) ** 2, -1, keepdims=True)
    return (x - m) / jnp.sqrt(v + 1e-5) * s + b


def _attend(q, k, v, dh):
    scale = dh ** -0.5
    outs = []
    for h in range(H):
        qh = q[:, h * dh:(h + 1) * dh]
        kh = k[:, h * dh:(h + 1) * dh]
        vh = v[:, h * dh:(h + 1) * dh]
        s = _dot_t(qh, kh) * scale
        rows = []
        for b in range(B):
            sb = s[b * S:(b + 1) * S, b * S:(b + 1) * S]
            sb = sb - jnp.max(sb, -1, keepdims=True)
            e = jnp.exp(sb)
            a = e / jnp.sum(e, -1, keepdims=True)
            rows.append(_dot(a, vh[b * S:(b + 1) * S, :]))
        outs.append(jnp.concatenate(rows, 0))
    return jnp.concatenate(outs, -1)


def _layer_body(xq_ref, xkv_ref, pv_ref, wq_ref, wk_ref, wv_ref,
                wo_ref, w1_ref, b1_ref, w2_ref, o_ref, *, dh):
    ls1 = pv_ref[0:1, :]
    lb1 = pv_ref[1:2, :]
    ls2 = pv_ref[2:3, :]
    lb2 = pv_ref[3:4, :]
    b2 = pv_ref[4:5, :]
    qn = _ln(xq_ref[:], ls1, lb1)
    kvn = _ln(xkv_ref[:], ls1, lb1)
    q = _dot(qn, wq_ref[:])
    k = _dot(kvn, wk_ref[:])
    v = _dot(kvn, wv_ref[:])
    o = _attend(q, k, v, dh)
    x = xq_ref[:] + _dot(o, wo_ref[:])
    xn = _ln(x, ls2, lb2)
    h = jnp.maximum(_dot(xn, w1_ref[:]) + b1_ref[:], 0.0)
    o_ref[:] = x + _dot(h, w2_ref[:]) + b2


def _layer_call(xq, xkv, p):
    d = xq.shape[-1]
    hid = p['W1'].shape[-1]
    pv = jnp.stack([p['ln1_s'], p['ln1_b'], p['ln2_s'], p['ln2_b'],
                    p['b2']], 0)
    return pl.pallas_call(
        functools.partial(_layer_body, dh=d // H),
        out_shape=jax.ShapeDtypeStruct((N, d), jnp.float32),
    )(xq, xkv, pv, p['Wq'], p['Wk'], p['Wv'], p['Wo'],
      p['W1'], p['b1'].reshape(1, hid), p['W2'])


def _qkv_body(x_ref, pv_ref, wq_ref, wk_ref, wv_ref,
              q_ref, k_ref, v_ref, xn_ref):
    t = pl.program_id(0)

    @pl.when(t == 0)
    def _():
        xn_ref[:] = _ln(x_ref[:], pv_ref[0:1, :], pv_ref[1:2, :])

    xn = xn_ref[:]
    q_ref[:] = _dot(xn, wq_ref[:])
    k_ref[:] = _dot(xn, wk_ref[:])
    v_ref[:] = _dot(xn, wv_ref[:])


def _qkv_call(x, p, ct=512):
    d = x.shape[-1]
    nt = pl.cdiv(d, ct)
    wspec = pl.BlockSpec((d, ct), lambda t: (0, t))
    ospec = pl.BlockSpec((N, ct), lambda t: (0, t))
    full = pl.BlockSpec((N, d), lambda t: (0, 0))
    pv = jnp.stack([p['ln1_s'], p['ln1_b']], 0)
    return pl.pallas_call(
        _qkv_body,
        grid=(nt,),
        in_specs=[full, pl.BlockSpec((2, d), lambda t: (0, 0)),
                  wspec, wspec, wspec],
        out_specs=(ospec, ospec, ospec),
        out_shape=tuple(jax.ShapeDtypeStruct((N, d), jnp.float32)
                        for _ in range(3)),
        scratch_shapes=[pltpu.VMEM((N, d), jnp.float32)],
    )(x, pv, p['Wq'], p['Wk'], p['Wv'])


def _attn_out_body(q_ref, k_ref, v_ref, wo_ref, res_ref, o_ref, *, dh):
    o = _attend(q_ref[:], k_ref[:], v_ref[:], dh)
    o_ref[:] = res_ref[:] + _dot(o, wo_ref[:])


def _attn_out_call(q, k, v, wo, res):
    d = q.shape[-1]
    return pl.pallas_call(
        functools.partial(_attn_out_body, dh=d // H),
        out_shape=jax.ShapeDtypeStruct((N, d), jnp.float32),
    )(q, k, v, wo, res)


def _ffn_body(x_ref, pv_ref, w1a_ref, w1b_ref, b1_ref, w2a_ref,
              w2b_ref, o_ref, xn_ref, *, ht, hid, nsub):
    t = pl.program_id(0)

    @pl.when(t == 0)
    def _():
        xn_ref[:] = _ln(x_ref[:], pv_ref[0:1, :], pv_ref[1:2, :])
        o_ref[:] = x_ref[:] + pv_ref[2:3, :]

    xn = xn_ref[:]
    acc = None
    for i, (w1_ref, w2_ref) in enumerate(((w1a_ref, w2a_ref),
                                          (w1b_ref, w2b_ref))):
        b1 = b1_ref[:, i * ht:(i + 1) * ht]
        h = jnp.maximum(_dot(xn, w1_ref[:]) + b1, 0.0)
        w2 = w2_ref[:]
        col = t * 2 * ht + i * ht + jax.lax.broadcasted_iota(
            jnp.int32, h.shape, 1)
        h = jnp.where(col < hid, h, 0.0)
        row = t * 2 * ht + i * ht + jax.lax.broadcasted_iota(
            jnp.int32, w2.shape, 0)
        w2 = jnp.where(row < hid, w2, 0.0)
        part = _dot(h, w2)
        acc = part if acc is None else acc + part
    o_ref[:] += acc


def _ffn_call(x, p, ht):
    d = x.shape[-1]
    hid = p['W1'].shape[-1]
    nsub = pl.cdiv(hid, ht)
    nt = pl.cdiv(nsub, 2)
    last = nsub - 1

    def wa(t):
        return (0, jnp.minimum(2 * t, last))

    def wb(t):
        return (0, jnp.minimum(2 * t + 1, last))

    def w2a(t):
        return (jnp.minimum(2 * t, last), 0)

    def w2b(t):
        return (jnp.minimum(2 * t + 1, last), 0)

    return pl.pallas_call(
        functools.partial(_ffn_body, ht=ht, hid=hid, nsub=nsub),
        grid=(nt,),
        in_specs=[
            pl.BlockSpec((N, d), lambda t: (0, 0)),
            pl.BlockSpec((3, d), lambda t: (0, 0)),
            pl.BlockSpec((d, ht), wa),
            pl.BlockSpec((d, ht), wb),
            pl.BlockSpec((1, 2 * ht), lambda t: (0, t)),
            pl.BlockSpec((ht, d), w2a),
            pl.BlockSpec((ht, d), w2b),
        ],
        out_specs=pl.BlockSpec((N, d), lambda t: (0, 0)),
        out_shape=jax.ShapeDtypeStruct((N, d), jnp.float32),
        scratch_shapes=[pltpu.VMEM((N, d), jnp.float32)],
    )(x, jnp.stack([p['ln2_s'], p['ln2_b'], p['b2']], 0),
      p['W1'], p['W1'], p['b1'].reshape(1, hid), p['W2'], p['W2'])


def _layer_big(xq, xkv, p):
    q, k, v = _qkv_call(xq, p)
    x = _attn_out_call(q, k, v, p['Wo'], xq)
    return _ffn_call(x, p, ht=768)


def _mm_body(*refs, relu, has_bias, has_res):
    x_ref, w_ref = refs[0], refs[1]
    rest = list(refs[2:-1])
    o_ref = refs[-1]
    acc = _dot(x_ref[:], w_ref[:])
    if has_bias:
        acc = acc + rest.pop(0)[:]
    if has_res:
        acc = acc + rest.pop(0)[:]
    if relu:
        acc = jnp.maximum(acc, 0.0)
    o_ref[:] = acc


def _mm_call(x, w, bias=None, res=None, relu=False, ct=None):
    d = x.shape[-1]
    nout = w.shape[-1]
    ct = ct or nout
    nt = pl.cdiv(nout, ct)
    args = [x, w]
    specs = [pl.BlockSpec((N, d), lambda t: (0, 0)),
             pl.BlockSpec((d, ct), lambda t: (0, t))]
    if bias is not None:
        args.append(bias.reshape(1, nout))
        specs.append(pl.BlockSpec((1, ct), lambda t: (0, t)))
    if res is not None:
        args.append(res)
        specs.append(pl.BlockSpec((N, ct), lambda t: (0, t)))
    return pl.pallas_call(
        functools.partial(_mm_body, relu=relu, has_bias=bias is not None,
                          has_res=res is not None),
        grid=(nt,),
        in_specs=specs,
        out_specs=pl.BlockSpec((N, ct), lambda t: (0, t)),
        out_shape=jax.ShapeDtypeStruct((N, nout), jnp.float32),
    )(*args)


def _layer(xq, xkv, p):
    if xq.shape[-1] <= 600:
        return _layer_call(xq, xq if xkv is None else xkv, p)
    return _layer_big(xq, xq if xkv is None else xkv, p)


def _encoder(x, kv, plist):
    for p in plist:
        x = _layer(x, kv, p)
    return x


def kernel(x_l, x_a, x_v, seq_lengths, params):
    P = params
    xl = x_l.reshape(N, -1)
    xa = x_a.reshape(N, -1)
    xv = x_v.reshape(N, -1)

    h_a_l = _encoder(xa, xl, P['a_l'])
    h_a_v = _encoder(xa, xv, P['a_v'])
    h_as = _encoder(jnp.concatenate([h_a_l, h_a_v], -1), None, P['a_mem'])
    h_v_l = _encoder(xv, xl, P['v_l'])
    h_v_a = _encoder(xv, xa, P['v_a'])
    h_vs = _encoder(jnp.concatenate([h_v_l, h_v_a], -1), None, P['v_mem'])

    x_l_ext = _mm_call(xl, P['W_lext'], bias=P['b_lext'])
    last_hs = jnp.concatenate([x_l_ext, h_as, h_vs], -1)
    z = _mm_call(last_hs, P['W_p1'], bias=P['b_p1'], relu=True, ct=512)
    z = _mm_call(z, P['W_p1'], bias=P['b_p1'], ct=512)
    proj = _encoder(z, None, P['last'])
    out = _mm_call(proj, P['W_out'], bias=P['b_out'])

    last_hs_proj = proj.reshape(B, S, -1).transpose(1, 0, 2)
    output = out.reshape(B, S, -1).transpose(1, 0, 2)
    return last_hs_proj, output
